# Initial kernel scaffold; baseline (speedup 1.0000x reference)
import jax
import jax.numpy as jnp
from jax.experimental import pallas as pl


def kernel(x1, x2, table1, table2, pe):
    raise NotImplementedError("write your pallas kernel here")


# speedup vs baseline: 3.9391x; 3.9391x over previous
import functools

import jax
import jax.numpy as jnp
from jax import lax
from jax.experimental import pallas as pl
from jax.experimental.pallas import tpu as pltpu
from jax.experimental.pallas import tpu_sc as plsc

_B = 4096
_S = 200
_HALF = 32
_EMBED = 64
_NC = 2
_NS = 16
_NW = _NC * _NS
_SEQ_PER_W = _B // _NW
_CH = 100
_L = 16


def _body(x1_hbm, x2_hbm, t1_hbm, t2_hbm, pe1_hbm, pe2_hbm, out_hbm,
          idx1_v, idx2_v, rows1_v, rows2_v, pe1_v, pe2_v, out_v, sem):
    wid = lax.axis_index("s") * _NC + lax.axis_index("c")
    base = wid * _SEQ_PER_W
    pltpu.sync_copy(pe1_hbm, pe1_v)
    pltpu.sync_copy(pe2_hbm, pe2_v)

    @pl.loop(0, _SEQ_PER_W)
    def _seq(i):
        b = base + i
        pltpu.sync_copy(x1_hbm.at[b], idx1_v)
        pltpu.sync_copy(x2_hbm.at[b], idx2_v)
        copies = []
        for j in range(2):
            dst = pl.ds(j * _CH, _CH)
            copies.append(pltpu.async_copy(t1_hbm.at[idx1_v.at[j]],
                                           rows1_v.at[dst], sem))
            copies.append(pltpu.async_copy(t2_hbm.at[idx2_v.at[j]],
                                           rows2_v.at[dst], sem))
        for c in copies:
            c.wait()

        @pl.loop(0, _S)
        def _row(r):
            for g in range(_HALF // _L):
                src = pl.ds(g * _L, _L)
                out_v[r, pl.ds(g * _L, _L)] = rows1_v[r, src] + pe1_v[r, src]
                out_v[r, pl.ds(_HALF + g * _L, _L)] = (
                    rows2_v[r, src] + pe2_v[r, src])

        pltpu.sync_copy(out_v, out_hbm.at[b])


@functools.partial(
    pl.kernel,
    out_type=jax.ShapeDtypeStruct((_B, _S, _EMBED), jnp.float32),
    mesh=plsc.VectorSubcoreMesh(core_axis_name="c", subcore_axis_name="s"),
    scratch_types=[
        pltpu.VMEM((2, _CH), jnp.int32),
        pltpu.VMEM((2, _CH), jnp.int32),
        pltpu.VMEM((_S, _HALF), jnp.float32),
        pltpu.VMEM((_S, _HALF), jnp.float32),
        pltpu.VMEM((_S, _HALF), jnp.float32),
        pltpu.VMEM((_S, _HALF), jnp.float32),
        pltpu.VMEM((_S, _EMBED), jnp.float32),
        pltpu.SemaphoreType.DMA,
    ],
    compiler_params=pltpu.CompilerParams(use_tc_tiling_on_sc=False),
)
def _emb_kernel(*refs):
    _body(*refs)


def kernel(x1, x2, table1, table2, pe):
    x1r = x1.astype(jnp.int32).reshape(_B, 2, _CH)
    x2r = x2.astype(jnp.int32).reshape(_B, 2, _CH)
    pe1 = pe[0, :_S, :_HALF]
    pe2 = pe[0, :_S, _HALF:]
    return _emb_kernel(x1r, x2r, table1, table2, pe1, pe2)


# speedup vs baseline: 5.1235x; 1.3007x over previous
import functools

import jax
import jax.numpy as jnp
from jax import lax
from jax.experimental import pallas as pl
from jax.experimental.pallas import tpu as pltpu
from jax.experimental.pallas import tpu_sc as plsc

_B = 4096
_S = 200
_HALF = 32
_EMBED = 64
_NC = 2
_NS = 16
_NW = _NC * _NS
_SEQ_PER_W = _B // _NW
_CH = 100
_L = 16
_NBUF = 2


def _issue_gather(t1, t2, idx1, idx2, r1, r2, s, sem):
    for j in range(2):
        dst = pl.ds(j * _CH, _CH)
        pltpu.async_copy(t1.at[idx1.at[s, j]], r1.at[dst], sem)
        pltpu.async_copy(t2.at[idx2.at[s, j]], r2.at[dst], sem)


def _wait_gather(t1, t2, idx1, idx2, r1, r2, s, sem):
    for j in range(2):
        dst = pl.ds(j * _CH, _CH)
        pltpu.make_async_copy(t1.at[idx1.at[s, j]], r1.at[dst], sem).wait()
        pltpu.make_async_copy(t2.at[idx2.at[s, j]], r2.at[dst], sem).wait()


def _compute(r1, r2, pe1_v, pe2_v, ob):
    @pl.loop(0, _S, unroll=2)
    def _row(r):
        for g in range(_HALF // _L):
            src = pl.ds(g * _L, _L)
            ob[r, pl.ds(g * _L, _L)] = r1[r, src] + pe1_v[r, src]
            ob[r, pl.ds(_HALF + g * _L, _L)] = r2[r, src] + pe2_v[r, src]


def _body(x1_hbm, x2_hbm, t1_hbm, t2_hbm, pe1_hbm, pe2_hbm, out_hbm,
          idx1_v, idx2_v, rows1_v, rows2_v, pe1_v, pe2_v, out_v,
          gsem0, gsem1, osem0, osem1):
    gsems = (gsem0, gsem1)
    osems = (osem0, osem1)
    wid = lax.axis_index("s") * _NC + lax.axis_index("c")
    base = wid * _SEQ_PER_W
    pltpu.sync_copy(pe1_hbm, pe1_v)
    pltpu.sync_copy(pe2_hbm, pe2_v)
    pltpu.sync_copy(x1_hbm.at[pl.ds(base, _SEQ_PER_W)], idx1_v)
    pltpu.sync_copy(x2_hbm.at[pl.ds(base, _SEQ_PER_W)], idx2_v)

    for b in range(_NBUF):
        _issue_gather(t1_hbm, t2_hbm, idx1_v, idx2_v,
                      rows1_v.at[b], rows2_v.at[b], b, gsems[b])

    @pl.loop(0, _SEQ_PER_W - _NBUF, step=_NBUF)
    def _main(i):
        for b in range(_NBUF):
            s = i + b
            _wait_gather(t1_hbm, t2_hbm, idx1_v, idx2_v,
                         rows1_v.at[b], rows2_v.at[b], s, gsems[b])

            @pl.when(i > 0)
            def _():
                pltpu.make_async_copy(out_v.at[b],
                                      out_hbm.at[base + s - _NBUF],
                                      osems[b]).wait()

            _compute(rows1_v.at[b], rows2_v.at[b], pe1_v, pe2_v, out_v.at[b])
            pltpu.async_copy(out_v.at[b], out_hbm.at[base + s], osems[b])
            _issue_gather(t1_hbm, t2_hbm, idx1_v, idx2_v,
                          rows1_v.at[b], rows2_v.at[b], s + _NBUF, gsems[b])

    for b in range(_NBUF):
        s = _SEQ_PER_W - _NBUF + b
        _wait_gather(t1_hbm, t2_hbm, idx1_v, idx2_v,
                     rows1_v.at[b], rows2_v.at[b], s, gsems[b])
        pltpu.make_async_copy(out_v.at[b], out_hbm.at[base + s - _NBUF],
                              osems[b]).wait()
        _compute(rows1_v.at[b], rows2_v.at[b], pe1_v, pe2_v, out_v.at[b])
        pltpu.async_copy(out_v.at[b], out_hbm.at[base + s], osems[b])
    for b in range(_NBUF):
        s = _SEQ_PER_W - _NBUF + b
        pltpu.make_async_copy(out_v.at[b], out_hbm.at[base + s],
                              osems[b]).wait()


@functools.partial(
    pl.kernel,
    out_type=jax.ShapeDtypeStruct((_B, _S, _EMBED), jnp.float32),
    mesh=plsc.VectorSubcoreMesh(core_axis_name="c", subcore_axis_name="s"),
    scratch_types=[
        pltpu.VMEM((_SEQ_PER_W, 2, _CH), jnp.int32),
        pltpu.VMEM((_SEQ_PER_W, 2, _CH), jnp.int32),
        pltpu.VMEM((_NBUF, _S, _HALF), jnp.float32),
        pltpu.VMEM((_NBUF, _S, _HALF), jnp.float32),
        pltpu.VMEM((_S, _HALF), jnp.float32),
        pltpu.VMEM((_S, _HALF), jnp.float32),
        pltpu.VMEM((_NBUF, _S, _EMBED), jnp.float32),
        pltpu.SemaphoreType.DMA,
        pltpu.SemaphoreType.DMA,
        pltpu.SemaphoreType.DMA,
        pltpu.SemaphoreType.DMA,
    ],
    compiler_params=pltpu.CompilerParams(use_tc_tiling_on_sc=False),
)
def _emb_kernel(*refs):
    _body(*refs)


def kernel(x1, x2, table1, table2, pe):
    x1r = x1.astype(jnp.int32).reshape(_B, 2, _CH)
    x2r = x2.astype(jnp.int32).reshape(_B, 2, _CH)
    pe1 = pe[0, :_S, :_HALF]
    pe2 = pe[0, :_S, _HALF:]
    return _emb_kernel(x1r, x2r, table1, table2, pe1, pe2)
